# baseline (speedup 1.0000x reference)
import functools

import jax
import jax.numpy as jnp
from jax import lax
from jax.experimental import pallas as pl
from jax.experimental.pallas import tpu as pltpu
from jax.experimental.pallas import tpu_sc as plsc

_NC = 2
_NS = 16
_C = 40
_ZR = 80
_BN = 2000
_BE = 4000


def _pack_bf16(acc):
  lo = lax.bitcast_convert_type(acc[:, :64].astype(jnp.bfloat16),
                                jnp.uint16).astype(jnp.uint32)
  hi = lax.bitcast_convert_type(acc[:, 64:].astype(jnp.bfloat16),
                                jnp.uint16).astype(jnp.uint32)
  return lax.bitcast_convert_type(lo | (hi << 16), jnp.int32)


def _proj_call(xs, w):
  j_dim, n, _ = xs.shape

  def body(xs_ref, w_ref, o_ref):
    acc = jnp.dot(xs_ref[0], w_ref[0], preferred_element_type=jnp.float32)
    for j in range(1, j_dim):
      acc += jnp.dot(xs_ref[j], w_ref[j], preferred_element_type=jnp.float32)
    o_ref[...] = acc

  return pl.pallas_call(
      body,
      grid=(n // _BN,),
      in_specs=[
          pl.BlockSpec((j_dim, _BN, 128), lambda i: (0, i, 0)),
          pl.BlockSpec((j_dim, 128, 128), lambda i: (0, 0, 0)),
      ],
      out_specs=pl.BlockSpec((_BN, 128), lambda i: (i, 0)),
      out_shape=jax.ShapeDtypeStruct((n, 128), jnp.float32),
  )(xs, w)


def _update_call(xs, agg, wux, wua, bu):
  j_dim, n, _ = xs.shape

  def body(xs_ref, agg_ref, wux_ref, wua_ref, bu_ref, o_ref):
    acc = jnp.dot(agg_ref[0] + agg_ref[1], wua_ref[...],
                  preferred_element_type=jnp.float32)
    for j in range(j_dim):
      acc += jnp.dot(xs_ref[j], wux_ref[j], preferred_element_type=jnp.float32)
    o_ref[...] = jnp.maximum(acc + bu_ref[...], 0.0)

  return pl.pallas_call(
      body,
      grid=(n // _BN,),
      in_specs=[
          pl.BlockSpec((j_dim, _BN, 128), lambda i: (0, i, 0)),
          pl.BlockSpec((_NC, _BN, 128), lambda i: (0, i, 0)),
          pl.BlockSpec((j_dim, 128, 128), lambda i: (0, 0, 0)),
          pl.BlockSpec((128, 128), lambda i: (0, 0)),
          pl.BlockSpec((1, 128), lambda i: (0, 0)),
      ],
      out_specs=pl.BlockSpec((_BN, 128), lambda i: (i, 0)),
      out_shape=jax.ShapeDtypeStruct((n, 128), jnp.float32),
  )(xs, agg, wux, wua, bu)


def _update_proj_call(xs, agg, wux, wua, bu, wnext):
  j_dim, n, _ = xs.shape

  def body(xs_ref, agg_ref, wux_ref, wua_ref, bu_ref, wn_ref, h_ref, p_ref):
    acc = jnp.dot(agg_ref[0] + agg_ref[1], wua_ref[...],
                  preferred_element_type=jnp.float32)
    for j in range(j_dim):
      acc += jnp.dot(xs_ref[j], wux_ref[j], preferred_element_type=jnp.float32)
    h = jnp.maximum(acc + bu_ref[...], 0.0)
    h_ref[...] = h
    p_ref[...] = jnp.dot(h, wn_ref[...], preferred_element_type=jnp.float32)

  return pl.pallas_call(
      body,
      grid=(n // _BN,),
      in_specs=[
          pl.BlockSpec((j_dim, _BN, 128), lambda i: (0, i, 0)),
          pl.BlockSpec((_NC, _BN, 128), lambda i: (0, i, 0)),
          pl.BlockSpec((j_dim, 128, 128), lambda i: (0, 0, 0)),
          pl.BlockSpec((128, 128), lambda i: (0, 0)),
          pl.BlockSpec((1, 128), lambda i: (0, 0)),
          pl.BlockSpec((128, 128), lambda i: (0, 0)),
      ],
      out_specs=[
          pl.BlockSpec((_BN, 128), lambda i: (i, 0)),
          pl.BlockSpec((_BN, 128), lambda i: (i, 0)),
      ],
      out_shape=[
          jax.ShapeDtypeStruct((n, 128), jnp.float32),
          jax.ShapeDtypeStruct((n, 128), jnp.float32),
      ],
  )(xs, agg, wux, wua, bu, wnext)


def _q_call(ea, w, b):
  e, de = ea.shape
  nl = w.shape[0]

  def body(ea_ref, w_ref, b_ref, o_ref):
    o_ref[0] = _pack_bf16(jnp.dot(ea_ref[...], w_ref[0],
                                  preferred_element_type=jnp.float32)
                          + b_ref[0])

  return pl.pallas_call(
      body,
      grid=(nl, e // _BE),
      in_specs=[
          pl.BlockSpec((_BE, de), lambda l, i: (i, 0)),
          pl.BlockSpec((1, de, 128), lambda l, i: (l, 0, 0)),
          pl.BlockSpec((1, 1, 128), lambda l, i: (l, 0, 0)),
      ],
      out_specs=pl.BlockSpec((1, _BE, 64), lambda l, i: (l, i, 0)),
      out_shape=jax.ShapeDtypeStruct((nl, e, 64), jnp.int32),
  )(ea, w, b)


def _sc_agg(p, q, layer, src3, dst3, zrows):
  n = p.shape[0]
  _, cpt, _ = dst3.shape
  e_per_tile = cpt * _C
  n_row_chunks = n // _ZR
  row_chunks_per_tile = -(-n_row_chunks // _NS)

  mesh = plsc.VectorSubcoreMesh(core_axis_name="c", subcore_axis_name="s")

  @functools.partial(
      pl.kernel,
      out_type=jax.ShapeDtypeStruct((_NC, n, 128), jnp.float32),
      mesh=mesh,
      compiler_params=pltpu.CompilerParams(needs_layout_passes=False),
      scratch_types=[
          pltpu.VMEM_SHARED((n, 128), jnp.float32),
          pltpu.VMEM((_ZR, 128), jnp.float32),
          pltpu.VMEM((e_per_tile,), jnp.int32),
          pltpu.VMEM((_C,), jnp.int32),
          pltpu.VMEM((_C,), jnp.int32),
          pltpu.VMEM((_C,), jnp.int32),
          pltpu.VMEM((_C, 128), jnp.float32),
          pltpu.VMEM((_C, 128), jnp.float32),
          pltpu.VMEM((_C, 128), jnp.float32),
          pltpu.VMEM((_C, 64), jnp.int32),
          pltpu.VMEM((_C, 64), jnp.int32),
          pltpu.VMEM((_C, 64), jnp.int32),
      ] + [pltpu.SemaphoreType.DMA] * 12,
  )
  def k(p_hbm, q_hbm, src_hbm, dst_hbm, z_hbm, out_hbm,
        agg_s, zb, src_v, d0, d1, d2, g0, g1, g2, q0, q1, q2,
        gs0, gs1, gs2, qs0, qs1, qs2, ds0, ds1, ds2, ss0, ss1, ss2):
    c = lax.axis_index("c")
    s = lax.axis_index("s")
    tid = s * _NC + c
    gath = (g0, g1, g2)
    qv = (q0, q1, q2)
    dv = (d0, d1, d2)
    gsem = (gs0, gs1, gs2)
    qsem = (qs0, qs1, qs2)
    dsem = (ds0, ds1, ds2)
    ssem = (ss0, ss1, ss2)

    pltpu.sync_copy(src_hbm.at[tid], src_v)

    pltpu.sync_copy(z_hbm, zb)
    for r in range(row_chunks_per_tile):
      rc = s + _NS * r

      @pl.when(rc < n_row_chunks)
      def _():
        base = pl.multiple_of(rc * _ZR, _ZR)
        pltpu.async_copy(zb, agg_s.at[pl.ds(base, _ZR)], gs0)

    for r in range(row_chunks_per_tile):
      rc = s + _NS * r

      @pl.when(rc < n_row_chunks)
      def _():
        base = pl.multiple_of(rc * _ZR, _ZR)
        pltpu.make_async_copy(zb, agg_s.at[pl.ds(base, _ZR)], gs0).wait()

    plsc.subcore_barrier()

    def drain_scatter(b):
      pltpu.make_async_copy(gath[b], agg_s.at[dv[b]], ssem[b]).wait()

    def start_fetch(j, b):
      sbase = pl.multiple_of(j * _C, _C)
      pltpu.async_copy(p_hbm.at[src_v.at[pl.ds(sbase, _C)]], gath[b], gsem[b])
      qbase = pl.multiple_of(tid * e_per_tile + j * _C, _C)
      pltpu.async_copy(q_hbm.at[layer, pl.ds(qbase, _C)], qv[b], qsem[b])
      pltpu.async_copy(dst_hbm.at[tid, j], dv[b], dsem[b])

    def process(j, b):
      sbase = pl.multiple_of(j * _C, _C)
      pltpu.make_async_copy(p_hbm.at[src_v.at[pl.ds(sbase, _C)]], gath[b],
                            gsem[b]).wait()
      qbase = pl.multiple_of(tid * e_per_tile + j * _C, _C)
      pltpu.make_async_copy(q_hbm.at[layer, pl.ds(qbase, _C)], qv[b],
                            qsem[b]).wait()
      pltpu.make_async_copy(dst_hbm.at[tid, j], dv[b], dsem[b]).wait()

      def row_body(r5, carry):
        for rr in range(8):
          row = r5 * 8 + rr
          for g in range(4):
            sl = pl.ds(g * 16, 16)
            sh = pl.ds(64 + g * 16, 16)
            wq = qv[b][row, sl]
            mask = jnp.int32(-65536)
            qlo = plsc.bitcast(wq << 16, jnp.float32)
            qhi = plsc.bitcast(wq & mask, jnp.float32)
            gath[b][row, sl] = jnp.maximum(gath[b][row, sl] + qlo, 0.0)
            gath[b][row, sh] = jnp.maximum(gath[b][row, sh] + qhi, 0.0)
        return carry

      lax.fori_loop(0, _C // 8, row_body, 0)
      pltpu.async_copy(gath[b], agg_s.at[dv[b]], ssem[b], add=True)

    start_fetch(0, 0)

    def step(j, b, drain):
      if drain:
        drain_scatter((j + 1) % 3)
      start_fetch(j + 1, (j + 1) % 3)
      process(j, b)

    step(0, 0, False)
    step(1, 1, False)

    def triple_body(m, carry):
      for t in range(3):
        j = 3 * m + 2 + t
        b = (2 + t) % 3

        @pl.when(j + 1 < cpt)
        def _():
          drain_scatter(t)
          start_fetch(j + 1, t)

        process(j, b)
      return carry

    lax.fori_loop(0, (cpt - 2) // 3, triple_body, 0)
    drain_scatter(0)
    drain_scatter(1)
    drain_scatter(2)

    plsc.subcore_barrier()

    for r in range(row_chunks_per_tile):
      rc = s + _NS * r

      @pl.when(rc < n_row_chunks)
      def _():
        base = pl.multiple_of(rc * _ZR, _ZR)
        rows = pl.ds(base, _ZR)
        pltpu.async_copy(agg_s.at[rows], out_hbm.at[c, rows], gs1)

    for r in range(row_chunks_per_tile):
      rc = s + _NS * r

      @pl.when(rc < n_row_chunks)
      def _():
        base = pl.multiple_of(rc * _ZR, _ZR)
        rows = pl.ds(base, _ZR)
        pltpu.make_async_copy(agg_s.at[rows], out_hbm.at[c, rows], gs1).wait()

  return k(p, q, src3, dst3, zrows)


def kernel(x, edge_index, edge_attr, params):
  e = edge_index.shape[1]
  cpt = e // (_NC * _NS) // _C
  src = edge_index[0].reshape(_NC * _NS, cpt * _C)
  dst = edge_index[1].reshape(_NC * _NS, cpt, _C)

  wmx, wq, bq, wux, wua, bus = [], [], [], [], [], []
  for (wm, bm, wu, bu) in params:
    in_c = wm.shape[0] - edge_attr.shape[1]
    wmx.append(wm[:in_c].reshape(in_c // 128, 128, 128))
    wq.append(wm[in_c:])
    bq.append(bm.reshape(1, -1))
    wux.append(wu[:in_c].reshape(in_c // 128, 128, 128))
    wua.append(wu[in_c:])
    bus.append(bu.reshape(1, -1))

  zrows = jnp.zeros((_ZR, 128), jnp.float32)

  q_all = _q_call(edge_attr, jnp.stack(wq), jnp.stack(bq))

  p = _proj_call(x[None], wmx[0])
  xin = x[None]
  hs = []
  for i in range(7):
    agg = _sc_agg(p, q_all, i, src, dst, zrows)
    if i < 6:
      h, p = _update_proj_call(xin, agg, wux[i], wua[i], bus[i], wmx[i + 1][0])
    else:
      h = _update_call(xin, agg, wux[i], wua[i], bus[i])
    hs.append(h)
    xin = h[None]

  hcat = jnp.stack(hs)
  p7 = _proj_call(hcat, wmx[7])
  agg = _sc_agg(p7, q_all, 7, src, dst, zrows)
  h7, p8 = _update_proj_call(hcat, agg, wux[7], wua[7], bus[7], wmx[8][0])

  agg = _sc_agg(p8, q_all, 8, src, dst, zrows)
  out = _update_call(h7[None], agg, wux[8], wua[8], bus[8])
  return out


# speedup vs baseline: 1.1668x; 1.1668x over previous
import functools

import jax
import jax.numpy as jnp
from jax import lax
from jax.experimental import pallas as pl
from jax.experimental.pallas import tpu as pltpu
from jax.experimental.pallas import tpu_sc as plsc

_NC = 2
_NS = 16
_C = 40
_ZR = 80
_BN = 2000
_BE = 4000


def _pack_bf16(acc):
  lo = lax.bitcast_convert_type(acc[:, :64].astype(jnp.bfloat16),
                                jnp.uint16).astype(jnp.uint32)
  hi = lax.bitcast_convert_type(acc[:, 64:].astype(jnp.bfloat16),
                                jnp.uint16).astype(jnp.uint32)
  return lax.bitcast_convert_type(lo | (hi << 16), jnp.int32)


def _proj_call(xs, w):
  j_dim, n, _ = xs.shape

  def body(xs_ref, w_ref, o_ref):
    acc = jnp.dot(xs_ref[0], w_ref[0], preferred_element_type=jnp.float32)
    for j in range(1, j_dim):
      acc += jnp.dot(xs_ref[j], w_ref[j], preferred_element_type=jnp.float32)
    o_ref[...] = acc

  return pl.pallas_call(
      body,
      grid=(n // _BN,),
      in_specs=[
          pl.BlockSpec((j_dim, _BN, 128), lambda i: (0, i, 0)),
          pl.BlockSpec((j_dim, 128, 128), lambda i: (0, 0, 0)),
      ],
      out_specs=pl.BlockSpec((_BN, 128), lambda i: (i, 0)),
      out_shape=jax.ShapeDtypeStruct((n, 128), jnp.float32),
  )(xs, w)


def _update_call(xs, agg, wux, wua, bu):
  j_dim, n, _ = xs.shape

  def body(xs_ref, agg_ref, wux_ref, wua_ref, bu_ref, o_ref):
    acc = jnp.dot(agg_ref[0] + agg_ref[1], wua_ref[...],
                  preferred_element_type=jnp.float32)
    for j in range(j_dim):
      acc += jnp.dot(xs_ref[j], wux_ref[j], preferred_element_type=jnp.float32)
    o_ref[...] = jnp.maximum(acc + bu_ref[...], 0.0)

  return pl.pallas_call(
      body,
      grid=(n // _BN,),
      in_specs=[
          pl.BlockSpec((j_dim, _BN, 128), lambda i: (0, i, 0)),
          pl.BlockSpec((_NC, _BN, 128), lambda i: (0, i, 0)),
          pl.BlockSpec((j_dim, 128, 128), lambda i: (0, 0, 0)),
          pl.BlockSpec((128, 128), lambda i: (0, 0)),
          pl.BlockSpec((1, 128), lambda i: (0, 0)),
      ],
      out_specs=pl.BlockSpec((_BN, 128), lambda i: (i, 0)),
      out_shape=jax.ShapeDtypeStruct((n, 128), jnp.float32),
  )(xs, agg, wux, wua, bu)


def _update_proj_call(xs, agg, wux, wua, bu, wnext):
  j_dim, n, _ = xs.shape

  def body(xs_ref, agg_ref, wux_ref, wua_ref, bu_ref, wn_ref, h_ref, p_ref):
    acc = jnp.dot(agg_ref[0] + agg_ref[1], wua_ref[...],
                  preferred_element_type=jnp.float32)
    for j in range(j_dim):
      acc += jnp.dot(xs_ref[j], wux_ref[j], preferred_element_type=jnp.float32)
    h = jnp.maximum(acc + bu_ref[...], 0.0)
    h_ref[...] = h
    p_ref[...] = jnp.dot(h, wn_ref[...], preferred_element_type=jnp.float32)

  return pl.pallas_call(
      body,
      grid=(n // _BN,),
      in_specs=[
          pl.BlockSpec((j_dim, _BN, 128), lambda i: (0, i, 0)),
          pl.BlockSpec((_NC, _BN, 128), lambda i: (0, i, 0)),
          pl.BlockSpec((j_dim, 128, 128), lambda i: (0, 0, 0)),
          pl.BlockSpec((128, 128), lambda i: (0, 0)),
          pl.BlockSpec((1, 128), lambda i: (0, 0)),
          pl.BlockSpec((128, 128), lambda i: (0, 0)),
      ],
      out_specs=[
          pl.BlockSpec((_BN, 128), lambda i: (i, 0)),
          pl.BlockSpec((_BN, 128), lambda i: (i, 0)),
      ],
      out_shape=[
          jax.ShapeDtypeStruct((n, 128), jnp.float32),
          jax.ShapeDtypeStruct((n, 128), jnp.float32),
      ],
  )(xs, agg, wux, wua, bu, wnext)


def _q_call(ea, w, b):
  e, de = ea.shape
  nl = w.shape[0]

  def body(ea_ref, w_ref, b_ref, o_ref):
    o_ref[0] = _pack_bf16(jnp.dot(ea_ref[...], w_ref[0],
                                  preferred_element_type=jnp.float32)
                          + b_ref[0])

  return pl.pallas_call(
      body,
      grid=(nl, e // _BE),
      in_specs=[
          pl.BlockSpec((_BE, de), lambda l, i: (i, 0)),
          pl.BlockSpec((1, de, 128), lambda l, i: (l, 0, 0)),
          pl.BlockSpec((1, 1, 128), lambda l, i: (l, 0, 0)),
      ],
      out_specs=pl.BlockSpec((1, _BE, 64), lambda l, i: (l, i, 0)),
      out_shape=jax.ShapeDtypeStruct((nl, e, 64), jnp.int32),
  )(ea, w, b)


def _q_call1(ea, w, b):
  return _q_call(ea, w[None], b[None])


def _sc_agg(p, q, layer, src3, dst3, zrows):
  n = p.shape[0]
  _, cpt, _ = dst3.shape
  e_per_tile = cpt * _C
  n_row_chunks = n // _ZR
  row_chunks_per_tile = -(-n_row_chunks // _NS)

  mesh = plsc.VectorSubcoreMesh(core_axis_name="c", subcore_axis_name="s")

  @functools.partial(
      pl.kernel,
      out_type=jax.ShapeDtypeStruct((_NC, n, 128), jnp.float32),
      mesh=mesh,
      compiler_params=pltpu.CompilerParams(needs_layout_passes=False),
      scratch_types=[
          pltpu.VMEM_SHARED((n, 128), jnp.float32),
          pltpu.VMEM((_ZR, 128), jnp.float32),
          pltpu.VMEM((e_per_tile,), jnp.int32),
          pltpu.VMEM((_C,), jnp.int32),
          pltpu.VMEM((_C,), jnp.int32),
          pltpu.VMEM((_C,), jnp.int32),
          pltpu.VMEM((_C, 128), jnp.float32),
          pltpu.VMEM((_C, 128), jnp.float32),
          pltpu.VMEM((_C, 128), jnp.float32),
          pltpu.VMEM((_C, 64), jnp.int32),
          pltpu.VMEM((_C, 64), jnp.int32),
          pltpu.VMEM((_C, 64), jnp.int32),
      ] + [pltpu.SemaphoreType.DMA] * 12,
  )
  def k(p_hbm, q_hbm, src_hbm, dst_hbm, z_hbm, out_hbm,
        agg_s, zb, src_v, d0, d1, d2, g0, g1, g2, q0, q1, q2,
        gs0, gs1, gs2, qs0, qs1, qs2, ds0, ds1, ds2, ss0, ss1, ss2):
    c = lax.axis_index("c")
    s = lax.axis_index("s")
    tid = s * _NC + c
    gath = (g0, g1, g2)
    qv = (q0, q1, q2)
    dv = (d0, d1, d2)
    gsem = (gs0, gs1, gs2)
    qsem = (qs0, qs1, qs2)
    dsem = (ds0, ds1, ds2)
    ssem = (ss0, ss1, ss2)

    pltpu.sync_copy(src_hbm.at[tid], src_v)

    pltpu.sync_copy(z_hbm, zb)
    for r in range(row_chunks_per_tile):
      rc = s + _NS * r

      @pl.when(rc < n_row_chunks)
      def _():
        base = pl.multiple_of(rc * _ZR, _ZR)
        pltpu.async_copy(zb, agg_s.at[pl.ds(base, _ZR)], gs0)

    for r in range(row_chunks_per_tile):
      rc = s + _NS * r

      @pl.when(rc < n_row_chunks)
      def _():
        base = pl.multiple_of(rc * _ZR, _ZR)
        pltpu.make_async_copy(zb, agg_s.at[pl.ds(base, _ZR)], gs0).wait()

    plsc.subcore_barrier()

    def drain_scatter(b):
      pltpu.make_async_copy(gath[b], agg_s.at[dv[b]], ssem[b]).wait()

    def start_fetch(j, b):
      sbase = pl.multiple_of(j * _C, _C)
      pltpu.async_copy(p_hbm.at[src_v.at[pl.ds(sbase, _C)]], gath[b], gsem[b])
      qbase = pl.multiple_of(tid * e_per_tile + j * _C, _C)
      pltpu.async_copy(q_hbm.at[layer, pl.ds(qbase, _C)], qv[b], qsem[b])
      pltpu.async_copy(dst_hbm.at[tid, j], dv[b], dsem[b])

    def process(j, b):
      sbase = pl.multiple_of(j * _C, _C)
      pltpu.make_async_copy(p_hbm.at[src_v.at[pl.ds(sbase, _C)]], gath[b],
                            gsem[b]).wait()
      qbase = pl.multiple_of(tid * e_per_tile + j * _C, _C)
      pltpu.make_async_copy(q_hbm.at[layer, pl.ds(qbase, _C)], qv[b],
                            qsem[b]).wait()
      pltpu.make_async_copy(dst_hbm.at[tid, j], dv[b], dsem[b]).wait()

      def row_body(r5, carry):
        for rr in range(8):
          row = r5 * 8 + rr
          for g in range(4):
            sl = pl.ds(g * 16, 16)
            sh = pl.ds(64 + g * 16, 16)
            wq = qv[b][row, sl]
            mask = jnp.int32(-65536)
            qlo = plsc.bitcast(wq << 16, jnp.float32)
            qhi = plsc.bitcast(wq & mask, jnp.float32)
            gath[b][row, sl] = jnp.maximum(gath[b][row, sl] + qlo, 0.0)
            gath[b][row, sh] = jnp.maximum(gath[b][row, sh] + qhi, 0.0)
        return carry

      lax.fori_loop(0, _C // 8, row_body, 0)
      pltpu.async_copy(gath[b], agg_s.at[dv[b]], ssem[b], add=True)

    start_fetch(0, 0)

    def step(j, b, drain):
      if drain:
        drain_scatter((j + 1) % 3)
      start_fetch(j + 1, (j + 1) % 3)
      process(j, b)

    step(0, 0, False)
    step(1, 1, False)

    def triple_body(m, carry):
      for t in range(3):
        j = 3 * m + 2 + t
        b = (2 + t) % 3

        @pl.when(j + 1 < cpt)
        def _():
          drain_scatter(t)
          start_fetch(j + 1, t)

        process(j, b)
      return carry

    lax.fori_loop(0, (cpt - 2) // 3, triple_body, 0)
    drain_scatter(0)
    drain_scatter(1)
    drain_scatter(2)

    plsc.subcore_barrier()

    for r in range(row_chunks_per_tile):
      rc = s + _NS * r

      @pl.when(rc < n_row_chunks)
      def _():
        base = pl.multiple_of(rc * _ZR, _ZR)
        rows = pl.ds(base, _ZR)
        pltpu.async_copy(agg_s.at[rows], out_hbm.at[c, rows], gs1)

    for r in range(row_chunks_per_tile):
      rc = s + _NS * r

      @pl.when(rc < n_row_chunks)
      def _():
        base = pl.multiple_of(rc * _ZR, _ZR)
        rows = pl.ds(base, _ZR)
        pltpu.make_async_copy(agg_s.at[rows], out_hbm.at[c, rows], gs1).wait()

  return k(p, q, src3, dst3, zrows)


def kernel(x, edge_index, edge_attr, params):
  e = edge_index.shape[1]
  cpt = e // (_NC * _NS) // _C
  src = edge_index[0].reshape(_NC * _NS, cpt * _C)
  dst = edge_index[1].reshape(_NC * _NS, cpt, _C)

  wmx, wq, bq, wux, wua, bus = [], [], [], [], [], []
  for (wm, bm, wu, bu) in params:
    in_c = wm.shape[0] - edge_attr.shape[1]
    wmx.append(wm[:in_c].reshape(in_c // 128, 128, 128))
    wq.append(wm[in_c:])
    bq.append(bm.reshape(1, -1))
    wux.append(wu[:in_c].reshape(in_c // 128, 128, 128))
    wua.append(wu[in_c:])
    bus.append(bu.reshape(1, -1))

  zrows = jnp.zeros((_ZR, 128), jnp.float32)

  qs = [_q_call1(edge_attr, wq[i], bq[i]) for i in range(9)]

  p = _proj_call(x[None], wmx[0])
  xin = x[None]
  hs = []
  for i in range(7):
    agg = _sc_agg(p, qs[i], 0, src, dst, zrows)
    if i < 6:
      h, p = _update_proj_call(xin, agg, wux[i], wua[i], bus[i], wmx[i + 1][0])
    else:
      h = _update_call(xin, agg, wux[i], wua[i], bus[i])
    hs.append(h)
    xin = h[None]

  hcat = jnp.stack(hs)
  p7 = _proj_call(hcat, wmx[7])
  agg = _sc_agg(p7, qs[7], 0, src, dst, zrows)
  h7, p8 = _update_proj_call(hcat, agg, wux[7], wua[7], bus[7], wmx[8][0])

  agg = _sc_agg(p8, qs[8], 0, src, dst, zrows)
  out = _update_call(h7[None], agg, wux[8], wua[8], bus[8])
  return out
